# baseline (speedup 1.0000x reference)
import functools

import jax
import jax.numpy as jnp
from jax.experimental import pallas as pl
from jax.experimental.pallas import tpu as pltpu


def _round_up(v, m):
    return (v + m - 1) // m * m


def _pad2d(a, rows, cols):
    if a.shape == (rows, cols):
        return a
    out = jnp.zeros((rows, cols), a.dtype)
    return out.at[: a.shape[0], : a.shape[1]].set(a)


def _fused_kernel(
    x_ref,
    adj_ref,
    w1_ref,
    b1_ref,
    w2_ref,
    b2_ref,
    o_ref,
    s1_scr,
    s2_scr,
    m_scr,
    *,
    n_lin,
    n_p1,
    n_p2,
    tl,
    tm,
    tp,
    nclass,
):
    i = pl.program_id(0)

    @pl.when(i < n_lin)
    def _():
        xb = x_ref[...].astype(jnp.bfloat16)
        wb = w1_ref[...].astype(jnp.bfloat16)
        s1 = jnp.dot(xb, wb, preferred_element_type=jnp.float32)
        s1_scr[pl.ds(i * tl, tl), :] = s1.astype(jnp.bfloat16)

    @pl.when((i >= n_lin) & (i < n_lin + n_p1))
    def _():
        j = i - n_lin
        a = adj_ref[...].astype(jnp.bfloat16)
        m_scr[pl.ds(j * tm, tm), :] = a
        y = jnp.dot(a, s1_scr[...], preferred_element_type=jnp.float32)
        h = jnp.maximum(y + b1_ref[...], 0.0).astype(jnp.bfloat16)
        wb = w2_ref[...].astype(jnp.bfloat16)
        s2 = jnp.dot(h, wb, preferred_element_type=jnp.float32)
        s2_scr[pl.ds(j * tm, tm), :] = s2.astype(jnp.bfloat16)

    @pl.when(i >= n_lin + n_p1)
    def _():
        k = i - n_lin - n_p1
        m = m_scr[pl.ds(k * tp, tp), :]
        y = jnp.dot(m, s2_scr[...], preferred_element_type=jnp.float32)
        logits = y + b2_ref[...]
        tpad, cpad = logits.shape
        if nclass < cpad:
            col = jax.lax.broadcasted_iota(jnp.int32, (tpad, cpad), 1)
            valid = col < nclass
            masked = jnp.where(valid, logits, jnp.float32(-1e30))
            mx = jnp.max(masked, axis=1, keepdims=True)
            z = masked - mx
            se = jnp.sum(
                jnp.where(valid, jnp.exp(z), 0.0), axis=1, keepdims=True
            )
        else:
            mx = jnp.max(logits, axis=1, keepdims=True)
            z = logits - mx
            se = jnp.sum(jnp.exp(z), axis=1, keepdims=True)
        o_ref[...] = z - jnp.log(se)


def _forward(x, adj, w1, b1, w2, b2, *, tile_m=512, tile_l=1024, tile_p=512):
    n, nfeat = x.shape
    nhid = w1.shape[1]
    nclass = w2.shape[1]

    half = max(8, _round_up((n + 1) // 2, 8))
    tm = min(tile_m, half)
    tl = min(tile_l, half)
    tp = min(tile_p, half)
    n_pad = _round_up(n, max(tm, tl, tp))
    f_pad = _round_up(nfeat, 128)
    h_pad = _round_up(nhid, 128)
    c_pad = _round_up(nclass, 128)

    xp = _pad2d(x, n_pad, f_pad)
    adjp = _pad2d(adj, n_pad, n_pad)
    w1p = _pad2d(w1, f_pad, h_pad)
    w2p = _pad2d(w2, h_pad, c_pad)
    b1p = _pad2d(b1, 1, h_pad)
    b2p = _pad2d(b2, 1, c_pad)

    n_lin = n_pad // tl
    n_p1 = n_pad // tm
    n_p2 = n_pad // tp
    grid = (n_lin + n_p1 + n_p2,)

    kern = functools.partial(
        _fused_kernel,
        n_lin=n_lin,
        n_p1=n_p1,
        n_p2=n_p2,
        tl=tl,
        tm=tm,
        tp=tp,
        nclass=nclass,
    )

    out = pl.pallas_call(
        kern,
        out_shape=jax.ShapeDtypeStruct((n_pad, c_pad), jnp.float32),
        grid=grid,
        in_specs=[
            pl.BlockSpec((tl, f_pad), lambda i: (jnp.minimum(i, n_lin - 1), 0)),
            pl.BlockSpec(
                (tm, n_pad), lambda i: (jnp.clip(i - n_lin, 0, n_p1 - 1), 0)
            ),
            pl.BlockSpec((f_pad, h_pad), lambda i: (0, 0)),
            pl.BlockSpec((1, h_pad), lambda i: (0, 0)),
            pl.BlockSpec((h_pad, c_pad), lambda i: (0, 0)),
            pl.BlockSpec((1, c_pad), lambda i: (0, 0)),
        ],
        out_specs=pl.BlockSpec(
            (tp, c_pad), lambda i: (jnp.clip(i - n_lin - n_p1, 0, n_p2 - 1), 0)
        ),
        scratch_shapes=[
            pltpu.VMEM((n_pad, h_pad), jnp.bfloat16),
            pltpu.VMEM((n_pad, c_pad), jnp.bfloat16),
            pltpu.VMEM((n_pad, n_pad), jnp.bfloat16),
        ],
        compiler_params=pltpu.CompilerParams(
            dimension_semantics=("arbitrary",)
        ),
    )(xp, adjp, w1p, b1p, w2p, b2p)

    if (n_pad, c_pad) != (n, nclass):
        out = out[:n, :nclass]
    return out


def kernel(x, adj, w1, b1, w2, b2):
    return _forward(x, adj, w1, b1, w2, b2)


# speedup vs baseline: 1.0422x; 1.0422x over previous
import functools

import jax
import jax.numpy as jnp
from jax.experimental import pallas as pl
from jax.experimental.pallas import tpu as pltpu


def _round_up(v, m):
    return (v + m - 1) // m * m


def _pad2d(a, rows, cols):
    if a.shape == (rows, cols):
        return a
    out = jnp.zeros((rows, cols), a.dtype)
    return out.at[: a.shape[0], : a.shape[1]].set(a)


def _fused_kernel(
    x_ref,
    adj_ref,
    w1_ref,
    b1_ref,
    w2_ref,
    b2_ref,
    o_ref,
    s1_scr,
    s2_scr,
    m_scr,
    *,
    n_lin,
    n_p1,
    n_p2,
    tl,
    tm,
    tp,
    nclass,
):
    i = pl.program_id(0)

    @pl.when(i < n_lin)
    def _():
        xb = x_ref[...].astype(jnp.bfloat16)
        wb = w1_ref[...].astype(jnp.bfloat16)
        s1 = jnp.dot(xb, wb, preferred_element_type=jnp.float32)
        s1_scr[pl.ds(i * tl, tl), :] = s1.astype(jnp.bfloat16)

    @pl.when((i >= n_lin) & (i < n_lin + n_p1))
    def _():
        j = i - n_lin
        a = adj_ref[...].astype(jnp.bfloat16)
        m_scr[pl.ds(j * tm, tm), :] = a
        y = jnp.dot(a, s1_scr[...], preferred_element_type=jnp.float32)
        h = jnp.maximum(y + b1_ref[...], 0.0).astype(jnp.bfloat16)
        wb = w2_ref[...].astype(jnp.bfloat16)
        s2 = jnp.dot(h, wb, preferred_element_type=jnp.float32)
        s2_scr[pl.ds(j * tm, tm), :] = s2.astype(jnp.bfloat16)

    @pl.when(i >= n_lin + n_p1)
    def _():
        k = i - n_lin - n_p1
        m = m_scr[pl.ds(k * tp, tp), :]
        y = jnp.dot(m, s2_scr[...], preferred_element_type=jnp.float32)
        logits = y + b2_ref[...]
        tpad, cpad = logits.shape
        if nclass < cpad:
            col = jax.lax.broadcasted_iota(jnp.int32, (tpad, cpad), 1)
            valid = col < nclass
            masked = jnp.where(valid, logits, jnp.float32(-1e30))
            mx = jnp.max(masked, axis=1, keepdims=True)
            z = masked - mx
            se = jnp.sum(
                jnp.where(valid, jnp.exp(z), 0.0), axis=1, keepdims=True
            )
        else:
            mx = jnp.max(logits, axis=1, keepdims=True)
            z = logits - mx
            se = jnp.sum(jnp.exp(z), axis=1, keepdims=True)
        o_ref[...] = z - jnp.log(se)


def _forward(x, adj, w1, b1, w2, b2, *, tile_m=512, tile_l=1024, tile_p=1024):
    n, nfeat = x.shape
    nhid = w1.shape[1]
    nclass = w2.shape[1]

    half = max(8, _round_up((n + 1) // 2, 8))
    tm = min(tile_m, half)
    tl = min(tile_l, half)
    tp = min(tile_p, half)
    n_pad = _round_up(n, max(tm, tl, tp))
    f_pad = _round_up(nfeat, 128)
    h_pad = _round_up(nhid, 128)
    c_pad = _round_up(nclass, 128)

    xp = _pad2d(x, n_pad, f_pad)
    adjp = _pad2d(adj, n_pad, n_pad)
    w1p = _pad2d(w1, f_pad, h_pad)
    w2p = _pad2d(w2, h_pad, c_pad)
    b1p = _pad2d(b1, 1, h_pad)
    b2p = _pad2d(b2, 1, c_pad)

    n_lin = n_pad // tl
    n_p1 = n_pad // tm
    n_p2 = n_pad // tp
    grid = (n_lin + n_p1 + n_p2,)

    kern = functools.partial(
        _fused_kernel,
        n_lin=n_lin,
        n_p1=n_p1,
        n_p2=n_p2,
        tl=tl,
        tm=tm,
        tp=tp,
        nclass=nclass,
    )

    out = pl.pallas_call(
        kern,
        out_shape=jax.ShapeDtypeStruct((n_pad, c_pad), jnp.float32),
        grid=grid,
        in_specs=[
            pl.BlockSpec((tl, f_pad), lambda i: (jnp.minimum(i, n_lin - 1), 0)),
            pl.BlockSpec(
                (tm, n_pad), lambda i: (jnp.clip(i - n_lin, 0, n_p1 - 1), 0)
            ),
            pl.BlockSpec((f_pad, h_pad), lambda i: (0, 0)),
            pl.BlockSpec((1, h_pad), lambda i: (0, 0)),
            pl.BlockSpec((h_pad, c_pad), lambda i: (0, 0)),
            pl.BlockSpec((1, c_pad), lambda i: (0, 0)),
        ],
        out_specs=pl.BlockSpec(
            (tp, c_pad), lambda i: (jnp.clip(i - n_lin - n_p1, 0, n_p2 - 1), 0)
        ),
        scratch_shapes=[
            pltpu.VMEM((n_pad, h_pad), jnp.bfloat16),
            pltpu.VMEM((n_pad, c_pad), jnp.bfloat16),
            pltpu.VMEM((n_pad, n_pad), jnp.bfloat16),
        ],
        compiler_params=pltpu.CompilerParams(
            dimension_semantics=("arbitrary",)
        ),
    )(xp, adjp, w1p, b1p, w2p, b2p)

    if (n_pad, c_pad) != (n, nclass):
        out = out[:n, :nclass]
    return out


def kernel(x, adj, w1, b1, w2, b2):
    return _forward(x, adj, w1, b1, w2, b2)
